# baseline (device time: 13974 ns/iter reference)
import jax
import jax.numpy as jnp
from jax import lax
from jax.experimental import pallas as pl
from jax.experimental.pallas import tpu as pltpu

M = 512
D = 512
HALF = M // 2
C = 4
R = HALF // C


def kernel(partial, resid, gamma):
    def body(
        p_ref,
        r_ref,
        g_ref,
        o_ref,
        x_send,
        x_recv,
        y_send,
        y_recv,
        x_send_sems,
        x_recv_sems,
        y_send_sems,
        y_recv_sems,
    ):
        my_x = lax.axis_index("x")
        my_y = lax.axis_index("y")
        x_peer = (1 - my_x, my_y)
        y_peer = (my_x, 1 - my_y)
        base = my_y * HALF
        other = (1 - my_y) * HALF

        barrier = pltpu.get_barrier_semaphore()
        for nbr in (x_peer, y_peer):
            pl.semaphore_signal(
                barrier, inc=1, device_id=nbr, device_id_type=pl.DeviceIdType.MESH
            )
        pl.semaphore_wait(barrier, 2)

        x_send[...] = p_ref[0, pl.ds(base, HALF), :].astype(jnp.bfloat16)

        x_rdmas = []
        for c in range(C):
            rd = pltpu.make_async_remote_copy(
                src_ref=x_send.at[pl.ds(c * R, R), :],
                dst_ref=x_recv.at[pl.ds(c * R, R), :],
                send_sem=x_send_sems.at[c],
                recv_sem=x_recv_sems.at[c],
                device_id=x_peer,
                device_id_type=pl.DeviceIdType.MESH,
            )
            rd.start()
            x_rdmas.append(rd)

        y_rdmas = []
        for c in range(C):
            x_rdmas[c].wait_recv()
            rows = pl.ds(base + c * R, R)
            y = (
                p_ref[0, rows, :]
                + x_recv[pl.ds(c * R, R), :].astype(jnp.float32)
                + r_ref[rows, :]
            )
            ms = jnp.mean(y * y, axis=-1, keepdims=True)
            out_c = y * lax.rsqrt(ms + 1e-6) * g_ref[...]
            o_ref[rows, :] = out_c
            y_send[pl.ds(c * R, R), :] = out_c.astype(jnp.bfloat16)
            rd = pltpu.make_async_remote_copy(
                src_ref=y_send.at[pl.ds(c * R, R), :],
                dst_ref=y_recv.at[pl.ds(c * R, R), :],
                send_sem=y_send_sems.at[c],
                recv_sem=y_recv_sems.at[c],
                device_id=y_peer,
                device_id_type=pl.DeviceIdType.MESH,
            )
            rd.start()
            y_rdmas.append(rd)

        for c in range(C):
            y_rdmas[c].wait_recv()
            o_ref[pl.ds(other + c * R, R), :] = y_recv[
                pl.ds(c * R, R), :
            ].astype(jnp.float32)

        for rd in x_rdmas:
            rd.wait_send()
        for rd in y_rdmas:
            rd.wait_send()

    return pl.pallas_call(
        body,
        out_shape=jax.ShapeDtypeStruct((M, D), jnp.float32),
        in_specs=[
            pl.BlockSpec(memory_space=pltpu.VMEM),
            pl.BlockSpec(memory_space=pltpu.VMEM),
            pl.BlockSpec(memory_space=pltpu.VMEM),
        ],
        out_specs=pl.BlockSpec(memory_space=pltpu.VMEM),
        scratch_shapes=[
            pltpu.VMEM((HALF, D), jnp.bfloat16),
            pltpu.VMEM((HALF, D), jnp.bfloat16),
            pltpu.VMEM((HALF, D), jnp.bfloat16),
            pltpu.VMEM((HALF, D), jnp.bfloat16),
            pltpu.SemaphoreType.DMA((C,)),
            pltpu.SemaphoreType.DMA((C,)),
            pltpu.SemaphoreType.DMA((C,)),
            pltpu.SemaphoreType.DMA((C,)),
        ],
        compiler_params=pltpu.CompilerParams(collective_id=0),
    )(partial, resid, gamma.reshape(1, D))


# device time: 12283 ns/iter; 1.1377x vs baseline; 1.1377x over previous
import jax
import jax.numpy as jnp
from jax import lax
from jax.experimental import pallas as pl
from jax.experimental.pallas import tpu as pltpu

M = 512
D = 512
HALF = M // 2
C = 8
R = HALF // C


def kernel(partial, resid, gamma):
    qb = (partial[0] + 0.5 * resid).astype(jnp.bfloat16)

    def body(q_ref, g_ref, o_ref, x_recv, x_send_sems, x_recv_sems,
             y_send_sems, y_recv_sems):
        my_x = lax.axis_index("x")
        my_y = lax.axis_index("y")
        x_peer = (1 - my_x, my_y)
        y_peer = (my_x, 1 - my_y)
        base = my_y * HALF

        barrier = pltpu.get_barrier_semaphore()
        for nbr in (x_peer, y_peer):
            pl.semaphore_signal(
                barrier, inc=1, device_id=nbr,
                device_id_type=pl.DeviceIdType.MESH,
            )
        pl.semaphore_wait(barrier, 2)

        x_rdmas = []
        for c in range(C):
            rd = pltpu.make_async_remote_copy(
                src_ref=q_ref.at[pl.ds(base + c * R, R), :],
                dst_ref=x_recv.at[pl.ds(c * R, R), :],
                send_sem=x_send_sems.at[c],
                recv_sem=x_recv_sems.at[c],
                device_id=x_peer,
                device_id_type=pl.DeviceIdType.MESH,
            )
            rd.start()
            x_rdmas.append(rd)

        y_rdmas = []
        for c in range(C):
            x_rdmas[c].wait_recv()
            rows = pl.ds(base + c * R, R)
            y = q_ref[rows, :].astype(jnp.float32) + x_recv[
                pl.ds(c * R, R), :
            ].astype(jnp.float32)
            ms = jnp.mean(y * y, axis=-1, keepdims=True)
            o_ref[rows, :] = (y * lax.rsqrt(ms + 1e-6) * g_ref[...]).astype(
                jnp.bfloat16
            )
            rd = pltpu.make_async_remote_copy(
                src_ref=o_ref.at[rows, :],
                dst_ref=o_ref.at[rows, :],
                send_sem=y_send_sems.at[c],
                recv_sem=y_recv_sems.at[c],
                device_id=y_peer,
                device_id_type=pl.DeviceIdType.MESH,
            )
            rd.start()
            y_rdmas.append(rd)

        for rd in y_rdmas:
            rd.wait_recv()
        for rd in x_rdmas:
            rd.wait_send()
        for rd in y_rdmas:
            rd.wait_send()

    return pl.pallas_call(
        body,
        out_shape=jax.ShapeDtypeStruct((M, D), jnp.bfloat16),
        in_specs=[pl.BlockSpec(memory_space=pltpu.VMEM)] * 2,
        out_specs=pl.BlockSpec(memory_space=pltpu.VMEM),
        scratch_shapes=[
            pltpu.VMEM((HALF, D), jnp.bfloat16),
            pltpu.SemaphoreType.DMA((C,)),
            pltpu.SemaphoreType.DMA((C,)),
            pltpu.SemaphoreType.DMA((C,)),
            pltpu.SemaphoreType.DMA((C,)),
        ],
        compiler_params=pltpu.CompilerParams(collective_id=0),
    )(qb, gamma.reshape(1, D))


# device time: 11858 ns/iter; 1.1784x vs baseline; 1.0358x over previous
import jax
import jax.numpy as jnp
from jax import lax
from jax.experimental import pallas as pl
from jax.experimental.pallas import tpu as pltpu

M = 512
D = 512
HALF = M // 2
C = 8
R = HALF // C


def kernel(partial, resid, gamma):
    row0 = lax.axis_index("y") * HALF
    qb = (
        lax.dynamic_slice(partial[0], (row0, 0), (HALF, D))
        + 0.5 * lax.dynamic_slice(resid, (row0, 0), (HALF, D))
    ).astype(jnp.bfloat16)

    def body(q_ref, g_ref, o_ref, x_recv, x_send_sems, x_recv_sems,
             y_send_sems, y_recv_sems):
        my_x = lax.axis_index("x")
        my_y = lax.axis_index("y")
        x_peer = (1 - my_x, my_y)
        y_peer = (my_x, 1 - my_y)
        base = my_y * HALF

        barrier = pltpu.get_barrier_semaphore()
        for nbr in (x_peer, y_peer):
            pl.semaphore_signal(
                barrier, inc=1, device_id=nbr,
                device_id_type=pl.DeviceIdType.MESH,
            )
        pl.semaphore_wait(barrier, 2)

        x_rdmas = []
        for c in range(C):
            rd = pltpu.make_async_remote_copy(
                src_ref=q_ref.at[pl.ds(c * R, R), :],
                dst_ref=x_recv.at[pl.ds(c * R, R), :],
                send_sem=x_send_sems.at[c],
                recv_sem=x_recv_sems.at[c],
                device_id=x_peer,
                device_id_type=pl.DeviceIdType.MESH,
            )
            rd.start()
            x_rdmas.append(rd)

        y_rdmas = []
        for c in range(C):
            x_rdmas[c].wait_recv()
            lrows = pl.ds(c * R, R)
            orows = pl.ds(base + c * R, R)
            y = q_ref[lrows, :].astype(jnp.float32) + x_recv[
                lrows, :
            ].astype(jnp.float32)
            ms = jnp.mean(y * y, axis=-1, keepdims=True)
            o_ref[orows, :] = (y * lax.rsqrt(ms + 1e-6) * g_ref[...]).astype(
                jnp.bfloat16
            )
            rd = pltpu.make_async_remote_copy(
                src_ref=o_ref.at[orows, :],
                dst_ref=o_ref.at[orows, :],
                send_sem=y_send_sems.at[c],
                recv_sem=y_recv_sems.at[c],
                device_id=y_peer,
                device_id_type=pl.DeviceIdType.MESH,
            )
            rd.start()
            y_rdmas.append(rd)

        for rd in y_rdmas:
            rd.wait_recv()
        for rd in x_rdmas:
            rd.wait_send()
        for rd in y_rdmas:
            rd.wait_send()

    return pl.pallas_call(
        body,
        out_shape=jax.ShapeDtypeStruct((M, D), jnp.bfloat16),
        in_specs=[pl.BlockSpec(memory_space=pltpu.VMEM)] * 2,
        out_specs=pl.BlockSpec(memory_space=pltpu.VMEM),
        scratch_shapes=[
            pltpu.VMEM((HALF, D), jnp.bfloat16),
            pltpu.SemaphoreType.DMA((C,)),
            pltpu.SemaphoreType.DMA((C,)),
            pltpu.SemaphoreType.DMA((C,)),
            pltpu.SemaphoreType.DMA((C,)),
        ],
        compiler_params=pltpu.CompilerParams(collective_id=0),
    )(qb, gamma.reshape(1, D))


# device time: 11805 ns/iter; 1.1837x vs baseline; 1.0045x over previous
import jax
import jax.numpy as jnp
from jax import lax
from jax.experimental import pallas as pl
from jax.experimental.pallas import tpu as pltpu

M = 512
D = 512
HALF = M // 2
C = 8
R = HALF // C


def kernel(partial, resid, gamma):
    row0 = lax.axis_index("y") * HALF
    qb = (
        lax.dynamic_slice(partial[0], (row0, 0), (HALF, D))
        + 0.5 * lax.dynamic_slice(resid, (row0, 0), (HALF, D))
    ).astype(jnp.bfloat16)
    return kernel_from_qb(qb, gamma)


def kernel_from_qb(qb, gamma):
    def body(q_ref, g_ref, o_ref, x_recv, x_send_sems, x_recv_sems,
             y_send_sems, y_recv_sems):
        my_x = lax.axis_index("x")
        my_y = lax.axis_index("y")
        x_peer = (1 - my_x, my_y)
        y_peer = (my_x, 1 - my_y)
        base = my_y * HALF

        barrier = pltpu.get_barrier_semaphore()
        for nbr in (x_peer, y_peer):
            pl.semaphore_signal(
                barrier, inc=1, device_id=nbr,
                device_id_type=pl.DeviceIdType.MESH,
            )
        pl.semaphore_wait(barrier, 2)

        x_rdmas = []
        for c in range(C):
            rd = pltpu.make_async_remote_copy(
                src_ref=q_ref.at[pl.ds(c * R, R), :],
                dst_ref=x_recv.at[pl.ds(c * R, R), :],
                send_sem=x_send_sems.at[c],
                recv_sem=x_recv_sems.at[c],
                device_id=x_peer,
                device_id_type=pl.DeviceIdType.MESH,
            )
            rd.start()
            x_rdmas.append(rd)

        y_rdmas = []
        for c in range(C):
            x_rdmas[c].wait_recv()
            lrows = pl.ds(c * R, R)
            orows = pl.ds(base + c * R, R)
            y = q_ref[lrows, :].astype(jnp.float32) + x_recv[
                lrows, :
            ].astype(jnp.float32)
            ms = jnp.mean(y * y, axis=-1, keepdims=True)
            o_ref[orows, :] = (y * lax.rsqrt(ms + 1e-6) * g_ref[...]).astype(
                jnp.bfloat16
            )
            rd = pltpu.make_async_remote_copy(
                src_ref=o_ref.at[orows, :],
                dst_ref=o_ref.at[orows, :],
                send_sem=y_send_sems.at[c],
                recv_sem=y_recv_sems.at[c],
                device_id=y_peer,
                device_id_type=pl.DeviceIdType.MESH,
            )
            rd.start()
            y_rdmas.append(rd)

        for rd in y_rdmas:
            rd.wait_recv()
        for rd in x_rdmas:
            rd.wait_send()
        for rd in y_rdmas:
            rd.wait_send()

    return pl.pallas_call(
        body,
        out_shape=jax.ShapeDtypeStruct((M, D), jnp.bfloat16),
        in_specs=[pl.BlockSpec(memory_space=pltpu.VMEM)] * 2,
        out_specs=pl.BlockSpec(memory_space=pltpu.VMEM),
        scratch_shapes=[
            pltpu.VMEM((HALF, D), jnp.bfloat16),
            pltpu.SemaphoreType.DMA((C,)),
            pltpu.SemaphoreType.DMA((C,)),
            pltpu.SemaphoreType.DMA((C,)),
            pltpu.SemaphoreType.DMA((C,)),
        ],
        compiler_params=pltpu.CompilerParams(collective_id=0),
    )(qb, gamma.reshape(1, D))
